# Initial kernel scaffold; baseline (speedup 1.0000x reference)
import jax
import jax.numpy as jnp
from jax.experimental import pallas as pl


def kernel(hidden_states, router_logits, w13_weight, w2_weight):
    raise NotImplementedError("write your pallas kernel here")


# speedup vs baseline: 1.9676x; 1.9676x over previous
import functools
import jax
import jax.numpy as jnp
from jax.experimental import pallas as pl
from jax.experimental.pallas import tpu as pltpu

_NUM_EXPERTS = 64
_TOP_K = 2
_HIDDEN = 1024
_INTER = 512
_TOKENS = 512


def _moe_dense_body(logits_ref, x_ref, w13_ref, w2_ref, out_ref):
    e = pl.program_id(0)

    logits = logits_ref[...]
    m1 = jnp.max(logits, axis=-1, keepdims=True)
    lane = jax.lax.broadcasted_iota(jnp.int32, logits.shape, 1)
    big = jnp.int32(10 ** 9)
    idx1 = jnp.min(jnp.where(logits == m1, lane, big), axis=-1, keepdims=True)
    masked = jnp.where(lane == idx1, -jnp.inf, logits)
    m2 = jnp.max(masked, axis=-1, keepdims=True)
    idx2 = jnp.min(jnp.where(masked == m2, lane, big), axis=-1, keepdims=True)
    w1 = 1.0 / (1.0 + jnp.exp(m2 - m1))
    w2w = 1.0 - w1
    col = jnp.where(idx1 == e, w1, 0.0) + jnp.where(idx2 == e, w2w, 0.0)

    x = x_ref[...]
    w13 = w13_ref[0]
    h = jax.lax.dot_general(x, w13, (((1,), (1,)), ((), ())),
                            preferred_element_type=jnp.float32)
    gate = h[:, :_INTER]
    up = h[:, _INTER:]
    act = gate * jax.nn.sigmoid(gate) * up
    w2 = w2_ref[0]
    o = jax.lax.dot_general(act, w2, (((1,), (1,)), ((), ())),
                            preferred_element_type=jnp.float32)

    @pl.when(e == 0)
    def _():
        out_ref[...] = jnp.zeros_like(out_ref)

    out_ref[...] += col * o


@jax.jit
def kernel(hidden_states, router_logits, w13_weight, w2_weight):
    grid = (_NUM_EXPERTS,)
    return pl.pallas_call(
        _moe_dense_body,
        grid=grid,
        in_specs=[
            pl.BlockSpec((_TOKENS, _NUM_EXPERTS), lambda e: (0, 0)),
            pl.BlockSpec((_TOKENS, _HIDDEN), lambda e: (0, 0)),
            pl.BlockSpec((1, 2 * _INTER, _HIDDEN), lambda e: (e, 0, 0)),
            pl.BlockSpec((1, _HIDDEN, _INTER), lambda e: (e, 0, 0)),
        ],
        out_specs=pl.BlockSpec((_TOKENS, _HIDDEN), lambda e: (0, 0)),
        out_shape=jax.ShapeDtypeStruct((_TOKENS, _HIDDEN), jnp.float32),
    )(router_logits, hidden_states, w13_weight, w2_weight)
